# baseline (speedup 1.0000x reference)
import functools

import jax
import jax.numpy as jnp
from jax import lax
from jax.experimental import pallas as pl
from jax.experimental.pallas import tpu as pltpu
from jax.experimental.pallas import tpu_sc as plsc

N = 10000
E = 320000
D = 128
A = 256
HOUT = 64

NC = 2
NS = 16
NW = NC * NS
BE = 128
KBUF = 4
GB = 64
EP = 327680
EPW = EP // NW
NBLKC = EPW // BE
NBLKG = EPW // GB
CHE = 2048
CHBG = CHE // GB
NCHW = EPW // CHE

NT = 10240
RPZ = NT // NS
ZR = 32

DUMMY = A
ASL = 384
RPT2 = ASL // NS
APT = A // NS

_MESH = dict(core_axis_name="c", subcore_axis_name="s")


@functools.partial(
    pl.kernel,
    out_type=jax.ShapeDtypeStruct((NC, NT, D), jnp.float32),
    mesh=plsc.VectorSubcoreMesh(**_MESH),
    compiler_params=pltpu.CompilerParams(needs_layout_passes=False),
    scratch_types=[
        pltpu.VMEM((EPW + KBUF * GB,), jnp.int32),
        pltpu.VMEM((CHE,), jnp.int32),
        pltpu.VMEM((16, D), jnp.float32),
        [pltpu.VMEM((GB, D), jnp.float32) for _ in range(KBUF)],
        pltpu.VMEM_SHARED((NT, D), jnp.float32),
        [pltpu.SemaphoreType.DMA for _ in range(KBUF)],
    ],
)
def _segsum_kernel(table, srcs, dsts, zeros, out, vsrc, vdst, vzero, bufs, acc, sems):
    cid = lax.axis_index("c")
    sid = lax.axis_index("s")
    wid = sid * NC + cid
    pltpu.sync_copy(zeros.at[pl.ds(0, 16)], vzero)
    for r in range(RPZ // 16):
        pltpu.sync_copy(vzero, acc.at[pl.ds(sid * RPZ + r * 16, 16)])
    pltpu.sync_copy(srcs.at[wid], vsrc.at[pl.ds(0, EPW)])
    zero16 = jnp.zeros((16,), jnp.int32)
    for v in range(KBUF * GB // 16):
        vsrc[pl.ds(EPW + v * 16, 16)] = zero16
    plsc.subcore_barrier()

    for t in range(KBUF):
        pltpu.async_copy(table.at[vsrc.at[pl.ds(t * GB, GB)]], bufs[t], sems[t])

    def chunk(c, carry):
        pltpu.sync_copy(dsts.at[wid, pl.ds(c * CHE, CHE)], vdst)
        for b in range(CHBG):
            t = b % KBUF
            j = c * CHBG + b
            pltpu.make_async_copy(
                table.at[vsrc.at[pl.ds(j * GB, GB)]], bufs[t], sems[t]).wait()
            pltpu.sync_copy(bufs[t], acc.at[vdst.at[pl.ds(b * GB, GB)]], add=True)
            pltpu.async_copy(
                table.at[vsrc.at[pl.ds((j + KBUF) * GB, GB)]], bufs[t], sems[t])
        return carry

    lax.fori_loop(0, NCHW, chunk, 0)
    for t in range(KBUF):
        pltpu.make_async_copy(
            table.at[vsrc.at[pl.ds(EPW + t * GB, GB)]], bufs[t], sems[t]).wait()
    plsc.subcore_barrier()
    pltpu.sync_copy(acc.at[pl.ds(sid * RPZ, RPZ)],
                    out.at[cid, pl.ds(sid * RPZ, RPZ)])


@functools.partial(
    pl.kernel,
    out_type=jax.ShapeDtypeStruct((3, A, D), jnp.float32),
    mesh=plsc.VectorSubcoreMesh(**_MESH),
    compiler_params=pltpu.CompilerParams(needs_layout_passes=False),
    scratch_types=[
        pltpu.VMEM((NBLKC, BE), jnp.int32),
        pltpu.VMEM((NBLKC, BE), jnp.int32),
        pltpu.VMEM((NT,), jnp.int32),
        pltpu.VMEM((A,), jnp.int32),
        pltpu.VMEM((APT,), jnp.int32),
        pltpu.VMEM((NBLKC * BE + 2 * BE,), jnp.int32),
        pltpu.VMEM((NBLKC * BE + 2 * BE,), jnp.int32),
        pltpu.VMEM((1, BE), jnp.int32),
        pltpu.VMEM((BE, D), jnp.float32),
        pltpu.VMEM((APT, D), jnp.float32),
        pltpu.VMEM_SHARED((ASL, D), jnp.float32),
        pltpu.SemaphoreType.DMA,
    ],
)
def _agent_conv_kernel(h, srcs, dsts, aidx, zeros, dslot, out,
                       src_v, dst_v, slot, aidx_v, aslot_v, csrc, cslot,
                       idx2d, rows, arow, acc, sem):
    cid = lax.axis_index("c")
    sid = lax.axis_index("s")
    wid = sid * NC + cid
    pltpu.sync_copy(zeros.at[pl.ds(0, RPT2)], acc.at[pl.ds(sid * RPT2, RPT2)])
    pltpu.sync_copy(dslot, slot)
    pltpu.sync_copy(aidx, aidx_v)
    for v in range(A // 16):
        a16 = aidx_v[pl.ds(v * 16, 16)]
        s16 = lax.iota(jnp.int32, 16) + v * 16
        plsc.store_scatter(slot, [a16], s16)
    pltpu.sync_copy(srcs.at[wid], src_v)
    pltpu.sync_copy(dsts.at[wid], dst_v)

    def cblk(j, cnt):
        for v in range(BE // 16):
            d16 = dst_v[j, pl.ds(v * 16, 16)]
            s16 = src_v[j, pl.ds(v * 16, 16)]
            f16 = plsc.load_gather(slot, [d16])
            m = f16 < DUMMY
            mi = m.astype(jnp.int32)
            pos = cnt + plsc.cumsum(mi) - mi
            plsc.store_scatter(csrc, [pos], s16, mask=m)
            plsc.store_scatter(cslot, [pos], f16, mask=m)
            cnt = cnt + jnp.sum(mi)
        return cnt

    cnt = lax.fori_loop(0, NBLKC, cblk, jnp.int32(0))
    zero16 = jnp.zeros((16,), jnp.int32)
    dummy16 = jnp.full((16,), DUMMY, jnp.int32)
    for v in range(BE // 16):
        csrc[pl.ds(cnt + v * 16, 16)] = zero16
        cslot[pl.ds(cnt + v * 16, 16)] = dummy16

    def gblk(b, carry):
        for v in range(BE // 16):
            idx2d[0, pl.ds(v * 16, 16)] = cslot[pl.ds(b * BE + v * 16, 16)]
        pltpu.async_copy(h.at[csrc.at[pl.ds(b * BE, BE)]], rows, sem).wait()
        pltpu.sync_copy(rows, acc.at[idx2d.at[0]], add=True)
        return carry

    nblk = (cnt + BE - 1) // BE
    lax.fori_loop(0, nblk, gblk, 0)
    plsc.subcore_barrier()

    a16 = aidx_v[pl.ds(sid * APT, 16)]
    aslot_v[pl.ds(0, 16)] = plsc.load_gather(slot, [a16])
    pltpu.async_copy(acc.at[aslot_v], arow, sem).wait()
    pltpu.sync_copy(arow, out.at[cid, pl.ds(sid * APT, APT)])

    @pl.when(cid == 0)
    def _():
        pltpu.async_copy(h.at[aidx_v.at[pl.ds(sid * APT, APT)]], arow, sem).wait()
        pltpu.sync_copy(arow, out.at[2, pl.ds(sid * APT, APT)])


def _dot_t(a, w):
    return lax.dot_general(a, w, (((1,), (1,)), ((), ())),
                           preferred_element_type=jnp.float32)


RB = 1024


def _tc_linear(partials, x, w_rel, b_rel, w_root):
    def body(p_ref, x_ref, wr_ref, br_ref, wo_ref, o_ref):
        agg = p_ref[0] + p_ref[1]
        y = _dot_t(agg, wr_ref[...]) + br_ref[...] + _dot_t(x_ref[...], wo_ref[...])
        o_ref[...] = jnp.maximum(y, 0.0)

    nb = NT // RB
    return pl.pallas_call(
        body,
        grid=(nb,),
        in_specs=[
            pl.BlockSpec((NC, RB, D), lambda i: (0, i, 0)),
            pl.BlockSpec((RB, D), lambda i: (i, 0)),
            pl.BlockSpec((D, D), lambda i: (0, 0)),
            pl.BlockSpec((1, D), lambda i: (0, 0)),
            pl.BlockSpec((D, D), lambda i: (0, 0)),
        ],
        out_specs=pl.BlockSpec((RB, D), lambda i: (i, 0)),
        out_shape=jax.ShapeDtypeStruct((NT, D), jnp.float32),
    )(partials, x, w_rel, b_rel, w_root)


def _tc_head(sel3, w_rel2, b_rel2, w_root2, wp1, bp1, wp2, bp2, wp3, bp3):
    def body(s_ref, wr, br, wo, w1, b1, w2, b2, w3, b3, o_ref):
        agg = s_ref[0] + s_ref[1]
        emb = jnp.maximum(_dot_t(agg, wr[...]) + br[...] + _dot_t(s_ref[2], wo[...]), 0.0)
        t = jnp.maximum(_dot_t(emb, w1[...]) + b1[...], 0.0)
        t = jnp.maximum(_dot_t(t, w2[...]) + b2[...], 0.0)
        o_ref[...] = _dot_t(t, w3[...]) + b3[...]

    return pl.pallas_call(
        body,
        out_shape=jax.ShapeDtypeStruct((A, HOUT), jnp.float32),
    )(sel3, w_rel2, b_rel2, w_root2, wp1, bp1, wp2, bp2, wp3, bp3)


def kernel(node_features, edge_index, agent_idx,
           W_rel1, b_rel1, W_root1,
           W_rel2, b_rel2, W_root2,
           Wp1, bp1, Wp2, bp2, Wp3, bp3):
    pad = EP - E
    src_f = jnp.concatenate([edge_index[0], jnp.zeros((pad,), jnp.int32)])
    dst_f = jnp.concatenate([edge_index[1], jnp.full((pad,), N, jnp.int32)])
    src_w = src_f.reshape(NW, EPW)
    dst_w = dst_f.reshape(NW, EPW)
    src_c = src_f.reshape(NW, NBLKC, BE)
    dst_c = dst_f.reshape(NW, NBLKC, BE)
    zeros = jnp.zeros((ZR, D), jnp.float32)
    dslot = jnp.full((NT,), DUMMY, jnp.int32)
    xp = jnp.pad(node_features, ((0, NT - N), (0, 0)))

    p1 = _segsum_kernel(node_features, src_w, dst_w, zeros)
    h = _tc_linear(p1, xp, W_rel1, b_rel1.reshape(1, D), W_root1)
    sel3 = _agent_conv_kernel(h, src_c, dst_c, agent_idx, zeros, dslot)
    out = _tc_head(sel3, W_rel2, b_rel2.reshape(1, D), W_root2,
                   Wp1, bp1.reshape(1, D), Wp2, bp2.reshape(1, D),
                   Wp3, bp3.reshape(1, HOUT))
    return out.reshape(A, 16, 4)


# speedup vs baseline: 1.2006x; 1.2006x over previous
import functools

import jax
import jax.numpy as jnp
from jax import lax
from jax.experimental import pallas as pl
from jax.experimental.pallas import tpu as pltpu
from jax.experimental.pallas import tpu_sc as plsc

N = 10000
E = 320000
D = 128
A = 256
HOUT = 64

NC = 2
NS = 16
NW = NC * NS
BE = 128
KBUF = 8
GD = 4
SD = 4
GB = 32
EP = 327680
EPW = EP // NW
NBLKC = EPW // BE
NBLKG = EPW // GB
CHE = 2048
CHBG = CHE // GB
NCHW = EPW // CHE

NT = 10240
RPZ = NT // NS
ZR = 32

DUMMY = A
ASL = 384
RPT2 = ASL // NS
APT = A // NS

_MESH = dict(core_axis_name="c", subcore_axis_name="s")


@functools.partial(
    pl.kernel,
    out_type=jax.ShapeDtypeStruct((NC, NT, D), jnp.float32),
    mesh=plsc.VectorSubcoreMesh(**_MESH),
    compiler_params=pltpu.CompilerParams(needs_layout_passes=False),
    scratch_types=[
        pltpu.VMEM((EPW + KBUF * GB,), jnp.int32),
        pltpu.VMEM((2, CHE), jnp.int32),
        pltpu.VMEM((8, D), jnp.float32),
        pltpu.VMEM((GB,), jnp.int32),
        [pltpu.VMEM((GB, D), jnp.float32) for _ in range(KBUF)],
        pltpu.VMEM_SHARED((NT, D), jnp.float32),
        [pltpu.SemaphoreType.DMA for _ in range(KBUF)],
        [pltpu.SemaphoreType.DMA for _ in range(KBUF)],
    ],
)
def _segsum_kernel(table, srcs, dsts, zeros, out,
                   vsrc, vdst, vzero, vdum, bufs, acc, sg, ss):
    cid = lax.axis_index("c")
    sid = lax.axis_index("s")
    wid = sid * NC + cid
    pltpu.sync_copy(zeros.at[pl.ds(0, 8)], vzero)
    for r in range(RPZ // 8):
        pltpu.sync_copy(vzero, acc.at[pl.ds(sid * RPZ + r * 8, 8)])
    pltpu.sync_copy(srcs.at[wid], vsrc.at[pl.ds(0, EPW)])
    zero16 = jnp.zeros((16,), jnp.int32)
    for v in range(KBUF * GB // 16):
        vsrc[pl.ds(EPW + v * 16, 16)] = zero16
    dum16 = jnp.full((16,), N, jnp.int32)
    for v in range(GB // 16):
        vdum[pl.ds(v * 16, 16)] = dum16
    plsc.subcore_barrier()

    for t in range(GD):
        pltpu.async_copy(table.at[vsrc.at[pl.ds(t * GB, GB)]], bufs[t], sg[t])
    for t in range(GD, KBUF):
        pltpu.async_copy(bufs[t], acc.at[vdum], ss[t], add=True)

    def chunk(c, carry):
        pltpu.sync_copy(dsts.at[wid, pl.ds(c * CHE, CHE)], vdst.at[c % 2])
        for b in range(CHBG):
            t = b % KBUF
            tp = (b + GD) % KBUF
            j = c * CHBG + b
            pltpu.make_async_copy(
                table.at[vsrc.at[pl.ds(j * GB, GB)]], bufs[t], sg[t]).wait()
            pltpu.async_copy(
                bufs[t], acc.at[vdst.at[c % 2, pl.ds(b * GB, GB)]], ss[t],
                add=True)
            pltpu.make_async_copy(bufs[tp], acc.at[vdum], ss[tp]).wait()
            pltpu.async_copy(
                table.at[vsrc.at[pl.ds((j + GD) * GB, GB)]], bufs[tp], sg[tp])
        return carry

    lax.fori_loop(0, NCHW, chunk, 0)
    for t in range(GD):
        pltpu.make_async_copy(
            table.at[vsrc.at[pl.ds((EPW // GB + t) * GB, GB)]],
            bufs[t], sg[t]).wait()
    for t in range(GD, KBUF):
        pltpu.make_async_copy(bufs[t], acc.at[vdum], ss[t]).wait()
    plsc.subcore_barrier()
    pltpu.sync_copy(acc.at[pl.ds(sid * RPZ, RPZ)],
                    out.at[cid, pl.ds(sid * RPZ, RPZ)])


@functools.partial(
    pl.kernel,
    out_type=jax.ShapeDtypeStruct((3, A, D), jnp.float32),
    mesh=plsc.VectorSubcoreMesh(**_MESH),
    compiler_params=pltpu.CompilerParams(needs_layout_passes=False),
    scratch_types=[
        pltpu.VMEM((NBLKC, BE), jnp.int32),
        pltpu.VMEM((NBLKC, BE), jnp.int32),
        pltpu.VMEM((NT,), jnp.int32),
        pltpu.VMEM((A,), jnp.int32),
        pltpu.VMEM((APT,), jnp.int32),
        pltpu.VMEM((NBLKC * BE + 2 * BE,), jnp.int32),
        pltpu.VMEM((NBLKC * BE + 2 * BE,), jnp.int32),
        pltpu.VMEM((1, BE), jnp.int32),
        pltpu.VMEM((BE, D), jnp.float32),
        pltpu.VMEM((APT, D), jnp.float32),
        pltpu.VMEM_SHARED((ASL, D), jnp.float32),
        pltpu.SemaphoreType.DMA,
    ],
)
def _agent_conv_kernel(h, srcs, dsts, aidx, zeros, dslot, out,
                       src_v, dst_v, slot, aidx_v, aslot_v, csrc, cslot,
                       idx2d, rows, arow, acc, sem):
    cid = lax.axis_index("c")
    sid = lax.axis_index("s")
    wid = sid * NC + cid
    pltpu.sync_copy(zeros.at[pl.ds(0, RPT2)], acc.at[pl.ds(sid * RPT2, RPT2)])
    pltpu.sync_copy(dslot, slot)
    pltpu.sync_copy(aidx, aidx_v)
    for v in range(A // 16):
        a16 = aidx_v[pl.ds(v * 16, 16)]
        s16 = lax.iota(jnp.int32, 16) + v * 16
        plsc.store_scatter(slot, [a16], s16)
    pltpu.sync_copy(srcs.at[wid], src_v)
    pltpu.sync_copy(dsts.at[wid], dst_v)

    def cblk(j, cnt):
        for v in range(BE // 16):
            d16 = dst_v[j, pl.ds(v * 16, 16)]
            s16 = src_v[j, pl.ds(v * 16, 16)]
            f16 = plsc.load_gather(slot, [d16])
            m = f16 < DUMMY
            mi = m.astype(jnp.int32)
            pos = cnt + plsc.cumsum(mi) - mi
            plsc.store_scatter(csrc, [pos], s16, mask=m)
            plsc.store_scatter(cslot, [pos], f16, mask=m)
            cnt = cnt + jnp.sum(mi)
        return cnt

    cnt = lax.fori_loop(0, NBLKC, cblk, jnp.int32(0))
    zero16 = jnp.zeros((16,), jnp.int32)
    dummy16 = jnp.full((16,), DUMMY, jnp.int32)
    for v in range(BE // 16):
        csrc[pl.ds(cnt + v * 16, 16)] = zero16
        cslot[pl.ds(cnt + v * 16, 16)] = dummy16

    def gblk(b, carry):
        for v in range(BE // 16):
            idx2d[0, pl.ds(v * 16, 16)] = cslot[pl.ds(b * BE + v * 16, 16)]
        pltpu.async_copy(h.at[csrc.at[pl.ds(b * BE, BE)]], rows, sem).wait()
        pltpu.sync_copy(rows, acc.at[idx2d.at[0]], add=True)
        return carry

    nblk = (cnt + BE - 1) // BE
    lax.fori_loop(0, nblk, gblk, 0)
    plsc.subcore_barrier()

    a16 = aidx_v[pl.ds(sid * APT, 16)]
    aslot_v[pl.ds(0, 16)] = plsc.load_gather(slot, [a16])
    pltpu.async_copy(acc.at[aslot_v], arow, sem).wait()
    pltpu.sync_copy(arow, out.at[cid, pl.ds(sid * APT, APT)])

    @pl.when(cid == 0)
    def _():
        pltpu.async_copy(h.at[aidx_v.at[pl.ds(sid * APT, APT)]], arow, sem).wait()
        pltpu.sync_copy(arow, out.at[2, pl.ds(sid * APT, APT)])


def _dot_t(a, w):
    return lax.dot_general(a, w, (((1,), (1,)), ((), ())),
                           preferred_element_type=jnp.float32)


RB = 1024


def _tc_linear(partials, x, w_rel, b_rel, w_root):
    def body(p_ref, x_ref, wr_ref, br_ref, wo_ref, o_ref):
        agg = p_ref[0] + p_ref[1]
        y = _dot_t(agg, wr_ref[...]) + br_ref[...] + _dot_t(x_ref[...], wo_ref[...])
        o_ref[...] = jnp.maximum(y, 0.0)

    nb = NT // RB
    return pl.pallas_call(
        body,
        grid=(nb,),
        in_specs=[
            pl.BlockSpec((NC, RB, D), lambda i: (0, i, 0)),
            pl.BlockSpec((RB, D), lambda i: (i, 0)),
            pl.BlockSpec((D, D), lambda i: (0, 0)),
            pl.BlockSpec((1, D), lambda i: (0, 0)),
            pl.BlockSpec((D, D), lambda i: (0, 0)),
        ],
        out_specs=pl.BlockSpec((RB, D), lambda i: (i, 0)),
        out_shape=jax.ShapeDtypeStruct((NT, D), jnp.float32),
    )(partials, x, w_rel, b_rel, w_root)


def _tc_head(sel3, w_rel2, b_rel2, w_root2, wp1, bp1, wp2, bp2, wp3, bp3):
    def body(s_ref, wr, br, wo, w1, b1, w2, b2, w3, b3, o_ref):
        agg = s_ref[0] + s_ref[1]
        emb = jnp.maximum(_dot_t(agg, wr[...]) + br[...] + _dot_t(s_ref[2], wo[...]), 0.0)
        t = jnp.maximum(_dot_t(emb, w1[...]) + b1[...], 0.0)
        t = jnp.maximum(_dot_t(t, w2[...]) + b2[...], 0.0)
        o_ref[...] = _dot_t(t, w3[...]) + b3[...]

    return pl.pallas_call(
        body,
        out_shape=jax.ShapeDtypeStruct((A, HOUT), jnp.float32),
    )(sel3, w_rel2, b_rel2, w_root2, wp1, bp1, wp2, bp2, wp3, bp3)


def kernel(node_features, edge_index, agent_idx,
           W_rel1, b_rel1, W_root1,
           W_rel2, b_rel2, W_root2,
           Wp1, bp1, Wp2, bp2, Wp3, bp3):
    pad = EP - E
    src_f = jnp.concatenate([edge_index[0], jnp.zeros((pad,), jnp.int32)])
    dst_f = jnp.concatenate([edge_index[1], jnp.full((pad,), N, jnp.int32)])
    src_w = src_f.reshape(NW, EPW)
    dst_w = dst_f.reshape(NW, EPW)
    src_c = src_f.reshape(NW, NBLKC, BE)
    dst_c = dst_f.reshape(NW, NBLKC, BE)
    zeros = jnp.zeros((ZR, D), jnp.float32)
    dslot = jnp.full((NT,), DUMMY, jnp.int32)
    xp = jnp.pad(node_features, ((0, NT - N), (0, 0)))

    p1 = _segsum_kernel(node_features, src_w, dst_w, zeros)
    h = _tc_linear(p1, xp, W_rel1, b_rel1.reshape(1, D), W_root1)
    sel3 = _agent_conv_kernel(h, src_c, dst_c, agent_idx, zeros, dslot)
    out = _tc_head(sel3, W_rel2, b_rel2.reshape(1, D), W_root2,
                   Wp1, bp1.reshape(1, D), Wp2, bp2.reshape(1, D),
                   Wp3, bp3.reshape(1, HOUT))
    return out.reshape(A, 16, 4)


# speedup vs baseline: 1.3478x; 1.1226x over previous
import functools

import jax
import jax.numpy as jnp
from jax import lax
from jax.experimental import pallas as pl
from jax.experimental.pallas import tpu as pltpu
from jax.experimental.pallas import tpu_sc as plsc

N = 10000
E = 320000
D = 128
A = 256
HOUT = 64

NC = 2
NS = 16
NW = NC * NS
BE = 128
KBUF = 8
GD = 2
SD = 6
GB = 32
EP = 327680
EPW = EP // NW
NBLKC = EPW // BE
NBLKG = EPW // GB
CHE = 2048
CHBG = CHE // GB
NCHW = EPW // CHE

NT = 10240
RPZ = NT // NS
ZR = 32

DUMMY = A
ASL = 384
RPT2 = ASL // NS
APT = A // NS

_MESH = dict(core_axis_name="c", subcore_axis_name="s")


@functools.partial(
    pl.kernel,
    out_type=jax.ShapeDtypeStruct((NC, NT, D), jnp.float32),
    mesh=plsc.VectorSubcoreMesh(**_MESH),
    compiler_params=pltpu.CompilerParams(needs_layout_passes=False),
    scratch_types=[
        pltpu.VMEM((EPW + KBUF * GB,), jnp.int32),
        pltpu.VMEM((2, CHE), jnp.int32),
        pltpu.VMEM((8, D), jnp.float32),
        pltpu.VMEM((GB,), jnp.int32),
        [pltpu.VMEM((GB, D), jnp.float32) for _ in range(KBUF)],
        pltpu.VMEM_SHARED((NT, D), jnp.float32),
        [pltpu.SemaphoreType.DMA for _ in range(KBUF)],
        [pltpu.SemaphoreType.DMA for _ in range(KBUF)],
    ],
)
def _segsum_kernel(table, srcs, dsts, zeros, out,
                   vsrc, vdst, vzero, vdum, bufs, acc, sg, ss):
    cid = lax.axis_index("c")
    sid = lax.axis_index("s")
    wid = sid * NC + cid
    pltpu.sync_copy(zeros.at[pl.ds(0, 8)], vzero)
    for r in range(RPZ // 8):
        pltpu.sync_copy(vzero, acc.at[pl.ds(sid * RPZ + r * 8, 8)])
    pltpu.sync_copy(srcs.at[wid], vsrc.at[pl.ds(0, EPW)])
    zero16 = jnp.zeros((16,), jnp.int32)
    for v in range(KBUF * GB // 16):
        vsrc[pl.ds(EPW + v * 16, 16)] = zero16
    dum16 = jnp.full((16,), N, jnp.int32)
    for v in range(GB // 16):
        vdum[pl.ds(v * 16, 16)] = dum16
    plsc.subcore_barrier()

    for t in range(GD):
        pltpu.async_copy(table.at[vsrc.at[pl.ds(t * GB, GB)]], bufs[t], sg[t])
    for t in range(GD, KBUF):
        pltpu.async_copy(bufs[t], acc.at[vdum], ss[t], add=True)

    def chunk(c, carry):
        pltpu.sync_copy(dsts.at[wid, pl.ds(c * CHE, CHE)], vdst.at[c % 2])
        for b in range(CHBG):
            t = b % KBUF
            tp = (b + GD) % KBUF
            j = c * CHBG + b
            pltpu.make_async_copy(
                table.at[vsrc.at[pl.ds(j * GB, GB)]], bufs[t], sg[t]).wait()
            pltpu.async_copy(
                bufs[t], acc.at[vdst.at[c % 2, pl.ds(b * GB, GB)]], ss[t],
                add=True)
            pltpu.make_async_copy(bufs[tp], acc.at[vdum], ss[tp]).wait()
            pltpu.async_copy(
                table.at[vsrc.at[pl.ds((j + GD) * GB, GB)]], bufs[tp], sg[tp])
        return carry

    lax.fori_loop(0, NCHW, chunk, 0)
    for t in range(GD):
        pltpu.make_async_copy(
            table.at[vsrc.at[pl.ds((EPW // GB + t) * GB, GB)]],
            bufs[t], sg[t]).wait()
    for t in range(GD, KBUF):
        pltpu.make_async_copy(bufs[t], acc.at[vdum], ss[t]).wait()
    plsc.subcore_barrier()
    pltpu.sync_copy(acc.at[pl.ds(sid * RPZ, RPZ)],
                    out.at[cid, pl.ds(sid * RPZ, RPZ)])


@functools.partial(
    pl.kernel,
    out_type=jax.ShapeDtypeStruct((3, A, D), jnp.float32),
    mesh=plsc.VectorSubcoreMesh(**_MESH),
    compiler_params=pltpu.CompilerParams(needs_layout_passes=False),
    scratch_types=[
        pltpu.VMEM((NBLKC, BE), jnp.int32),
        pltpu.VMEM((NBLKC, BE), jnp.int32),
        pltpu.VMEM((NT,), jnp.int32),
        pltpu.VMEM((A,), jnp.int32),
        pltpu.VMEM((APT,), jnp.int32),
        pltpu.VMEM((NBLKC * BE + 2 * BE,), jnp.int32),
        pltpu.VMEM((NBLKC * BE + 2 * BE,), jnp.int32),
        pltpu.VMEM((1, BE), jnp.int32),
        pltpu.VMEM((BE, D), jnp.float32),
        pltpu.VMEM((APT, D), jnp.float32),
        pltpu.VMEM_SHARED((ASL, D), jnp.float32),
        pltpu.SemaphoreType.DMA,
    ],
)
def _agent_conv_kernel(h, srcs, dsts, aidx, zeros, dslot, out,
                       src_v, dst_v, slot, aidx_v, aslot_v, csrc, cslot,
                       idx2d, rows, arow, acc, sem):
    cid = lax.axis_index("c")
    sid = lax.axis_index("s")
    wid = sid * NC + cid
    pltpu.sync_copy(zeros.at[pl.ds(0, RPT2)], acc.at[pl.ds(sid * RPT2, RPT2)])
    pltpu.sync_copy(dslot, slot)
    pltpu.sync_copy(aidx, aidx_v)
    for v in range(A // 16):
        a16 = aidx_v[pl.ds(v * 16, 16)]
        s16 = lax.iota(jnp.int32, 16) + v * 16
        plsc.store_scatter(slot, [a16], s16)
    pltpu.sync_copy(srcs.at[wid], src_v)
    pltpu.sync_copy(dsts.at[wid], dst_v)

    def cblk(j, cnt):
        for v in range(BE // 16):
            d16 = dst_v[j, pl.ds(v * 16, 16)]
            s16 = src_v[j, pl.ds(v * 16, 16)]
            f16 = plsc.load_gather(slot, [d16])
            m = f16 < DUMMY
            mi = m.astype(jnp.int32)
            pos = cnt + plsc.cumsum(mi) - mi
            plsc.store_scatter(csrc, [pos], s16, mask=m)
            plsc.store_scatter(cslot, [pos], f16, mask=m)
            cnt = cnt + jnp.sum(mi)
        return cnt

    cnt = lax.fori_loop(0, NBLKC, cblk, jnp.int32(0))
    zero16 = jnp.zeros((16,), jnp.int32)
    dummy16 = jnp.full((16,), DUMMY, jnp.int32)
    for v in range(BE // 16):
        csrc[pl.ds(cnt + v * 16, 16)] = zero16
        cslot[pl.ds(cnt + v * 16, 16)] = dummy16

    def gblk(b, carry):
        for v in range(BE // 16):
            idx2d[0, pl.ds(v * 16, 16)] = cslot[pl.ds(b * BE + v * 16, 16)]
        pltpu.async_copy(h.at[csrc.at[pl.ds(b * BE, BE)]], rows, sem).wait()
        pltpu.sync_copy(rows, acc.at[idx2d.at[0]], add=True)
        return carry

    nblk = (cnt + BE - 1) // BE
    lax.fori_loop(0, nblk, gblk, 0)
    plsc.subcore_barrier()

    a16 = aidx_v[pl.ds(sid * APT, 16)]
    aslot_v[pl.ds(0, 16)] = plsc.load_gather(slot, [a16])
    pltpu.async_copy(acc.at[aslot_v], arow, sem).wait()
    pltpu.sync_copy(arow, out.at[cid, pl.ds(sid * APT, APT)])

    @pl.when(cid == 0)
    def _():
        pltpu.async_copy(h.at[aidx_v.at[pl.ds(sid * APT, APT)]], arow, sem).wait()
        pltpu.sync_copy(arow, out.at[2, pl.ds(sid * APT, APT)])


def _dot_t(a, w):
    return lax.dot_general(a, w, (((1,), (1,)), ((), ())),
                           preferred_element_type=jnp.float32)


RB = 1024


def _tc_linear(partials, x, w_rel, b_rel, w_root):
    def body(p_ref, x_ref, wr_ref, br_ref, wo_ref, o_ref):
        agg = p_ref[0] + p_ref[1]
        y = _dot_t(agg, wr_ref[...]) + br_ref[...] + _dot_t(x_ref[...], wo_ref[...])
        o_ref[...] = jnp.maximum(y, 0.0)

    nb = NT // RB
    return pl.pallas_call(
        body,
        grid=(nb,),
        in_specs=[
            pl.BlockSpec((NC, RB, D), lambda i: (0, i, 0)),
            pl.BlockSpec((RB, D), lambda i: (i, 0)),
            pl.BlockSpec((D, D), lambda i: (0, 0)),
            pl.BlockSpec((1, D), lambda i: (0, 0)),
            pl.BlockSpec((D, D), lambda i: (0, 0)),
        ],
        out_specs=pl.BlockSpec((RB, D), lambda i: (i, 0)),
        out_shape=jax.ShapeDtypeStruct((NT, D), jnp.float32),
    )(partials, x, w_rel, b_rel, w_root)


def _tc_head(sel3, w_rel2, b_rel2, w_root2, wp1, bp1, wp2, bp2, wp3, bp3):
    def body(s_ref, wr, br, wo, w1, b1, w2, b2, w3, b3, o_ref):
        agg = s_ref[0] + s_ref[1]
        emb = jnp.maximum(_dot_t(agg, wr[...]) + br[...] + _dot_t(s_ref[2], wo[...]), 0.0)
        t = jnp.maximum(_dot_t(emb, w1[...]) + b1[...], 0.0)
        t = jnp.maximum(_dot_t(t, w2[...]) + b2[...], 0.0)
        o_ref[...] = _dot_t(t, w3[...]) + b3[...]

    return pl.pallas_call(
        body,
        out_shape=jax.ShapeDtypeStruct((A, HOUT), jnp.float32),
    )(sel3, w_rel2, b_rel2, w_root2, wp1, bp1, wp2, bp2, wp3, bp3)


def kernel(node_features, edge_index, agent_idx,
           W_rel1, b_rel1, W_root1,
           W_rel2, b_rel2, W_root2,
           Wp1, bp1, Wp2, bp2, Wp3, bp3):
    pad = EP - E
    src_f = jnp.concatenate([edge_index[0], jnp.zeros((pad,), jnp.int32)])
    dst_f = jnp.concatenate([edge_index[1], jnp.full((pad,), N, jnp.int32)])
    src_w = src_f.reshape(NW, EPW)
    dst_w = dst_f.reshape(NW, EPW)
    src_c = src_f.reshape(NW, NBLKC, BE)
    dst_c = dst_f.reshape(NW, NBLKC, BE)
    zeros = jnp.zeros((ZR, D), jnp.float32)
    dslot = jnp.full((NT,), DUMMY, jnp.int32)
    xp = jnp.pad(node_features, ((0, NT - N), (0, 0)))

    p1 = _segsum_kernel(node_features, src_w, dst_w, zeros)
    h = _tc_linear(p1, xp, W_rel1, b_rel1.reshape(1, D), W_root1)
    sel3 = _agent_conv_kernel(h, src_c, dst_c, agent_idx, zeros, dslot)
    out = _tc_head(sel3, W_rel2, b_rel2.reshape(1, D), W_root2,
                   Wp1, bp1.reshape(1, D), Wp2, bp2.reshape(1, D),
                   Wp3, bp3.reshape(1, HOUT))
    return out.reshape(A, 16, 4)
